# baseline (speedup 1.0000x reference)
import jax
import jax.numpy as jnp
from jax import lax
from jax.experimental import pallas as pl
from jax.experimental.pallas import tpu as pltpu
from jax.experimental.pallas import tpu_sc as plsc

VOCAB = 100000
HIDDEN = 64
B = 4096
C = 50
T = 50

NC = 2
NS = 16
NW = NC * NS
BPW = B // NW
NP = BPW // 2

CP = 56
TP = 56
TG = 4
TPAD = TG * 16
CP2 = 2 * CP
TP2 = 2 * TP
TROWS = TP + TPAD


def _cbow_body(ctx_hbm, tid_hbm, in_emb_hbm, out_emb_hbm, y_hbm,
               ctx_v, tid_v, cr_a, cr_b, tr_a, tr_b, out_v,
               sem_ci, sem_ti, sem_ca, sem_cb, sem_ta, sem_tb, sem_out):
    wid = lax.axis_index("s") * NC + lax.axis_index("c")
    base = wid * BPW
    lane = lax.iota(jnp.int32, 16)

    pltpu.async_copy(ctx_hbm.at[pl.ds(wid * NP, NP)], ctx_v, sem_ci)
    pltpu.async_copy(tid_hbm.at[pl.ds(wid * NP, NP)], tid_v, sem_ti)
    pltpu.make_async_copy(ctx_hbm.at[pl.ds(wid * NP, NP)], ctx_v, sem_ci).wait()
    pltpu.make_async_copy(tid_hbm.at[pl.ds(wid * NP, NP)], tid_v, sem_ti).wait()

    def fire(p, crows, trows, sem_c, sem_t):
        pltpu.async_copy(in_emb_hbm.at[ctx_v.at[p]], crows, sem_c)
        pltpu.async_copy(out_emb_hbm.at[tid_v.at[p]], trows.at[pl.ds(0, TP2)],
                         sem_t)

    def drain(p, crows, trows, sem_c, sem_t):
        pltpu.make_async_copy(in_emb_hbm.at[ctx_v.at[p]], crows, sem_c).wait()
        pltpu.make_async_copy(out_emb_hbm.at[tid_v.at[p]],
                              trows.at[pl.ds(0, TP2)], sem_t).wait()

    def compute_one(i, crows, trows, cbase, tbase):
        def pool(c, accs):
            a0, a1, a2, a3 = accs
            cc = cbase + c
            a0 = a0 + crows[cc, pl.ds(0, 16)]
            a1 = a1 + crows[cc, pl.ds(16, 16)]
            a2 = a2 + crows[cc, pl.ds(32, 16)]
            a3 = a3 + crows[cc, pl.ds(48, 16)]
            return (a0, a1, a2, a3)

        z = jnp.zeros((16,), jnp.float32)
        con0, con1, con2, con3 = lax.fori_loop(0, C, pool, (z, z, z, z))

        def sgroup(g, carry2):
            tb = tbase + g * 16
            acc = jnp.zeros((16,), jnp.float32)
            for k in range(16):
                tt = tb + k
                p = trows[tt, pl.ds(0, 16)] * con0
                p = p + trows[tt, pl.ds(16, 16)] * con1
                p = p + trows[tt, pl.ds(32, 16)] * con2
                p = p + trows[tt, pl.ds(48, 16)] * con3
                for sh in (8, 4, 2, 1):
                    p = p + p.at[lane ^ sh].get(mode="promise_in_bounds")
                acc = jnp.where(lane == k, p, acc)
            out_v[i, pl.ds(g * 16, 16)] = acc
            return carry2

        lax.fori_loop(0, TG, sgroup, 0)

    def compute_pair(p, crows, trows):
        compute_one(2 * p, crows, trows, 0, 0)
        compute_one(2 * p + 1, crows, trows, CP, TP)

    fire(0, cr_a, tr_a, sem_ca, sem_ta)

    def step(j, carry):
        p0 = 2 * j
        fire(p0 + 1, cr_b, tr_b, sem_cb, sem_tb)
        drain(p0, cr_a, tr_a, sem_ca, sem_ta)
        compute_pair(p0, cr_a, tr_a)
        pnext = jnp.minimum(p0 + 2, NP - 1)
        fire(pnext, cr_a, tr_a, sem_ca, sem_ta)
        drain(p0 + 1, cr_b, tr_b, sem_cb, sem_tb)
        compute_pair(p0 + 1, cr_b, tr_b)
        return carry

    lax.fori_loop(0, NP // 2, step, 0)
    drain(NP - 1, cr_a, tr_a, sem_ca, sem_ta)

    pltpu.async_copy(out_v, y_hbm.at[pl.ds(base, BPW)], sem_out)
    pltpu.make_async_copy(out_v, y_hbm.at[pl.ds(base, BPW)], sem_out).wait()


@jax.jit
def _cbow_sc(ctx_pad, tid_pad, in_emb, out_emb):
    mesh = plsc.VectorSubcoreMesh(core_axis_name="c", subcore_axis_name="s")
    f = pl.kernel(
        _cbow_body,
        out_type=jax.ShapeDtypeStruct((B, TPAD), jnp.float32),
        mesh=mesh,
        scratch_types=[
            pltpu.VMEM((NP, CP2), jnp.int32),
            pltpu.VMEM((NP, TP2), jnp.int32),
            pltpu.VMEM((CP2, HIDDEN), jnp.float32),
            pltpu.VMEM((CP2, HIDDEN), jnp.float32),
            pltpu.VMEM((TROWS, HIDDEN), jnp.float32),
            pltpu.VMEM((TROWS, HIDDEN), jnp.float32),
            pltpu.VMEM((BPW, TPAD), jnp.float32),
            pltpu.SemaphoreType.DMA,
            pltpu.SemaphoreType.DMA,
            pltpu.SemaphoreType.DMA,
            pltpu.SemaphoreType.DMA,
            pltpu.SemaphoreType.DMA,
            pltpu.SemaphoreType.DMA,
            pltpu.SemaphoreType.DMA,
        ],
        compiler_params=pltpu.CompilerParams(use_tc_tiling_on_sc=False),
    )
    return f(ctx_pad, tid_pad, in_emb, out_emb)


def kernel(contexts, t, in_emb, out_emb):
    contexts = contexts.astype(jnp.int32)
    t = t.astype(jnp.int32)
    ctx_pad = jnp.pad(contexts, ((0, 0), (0, CP - C))).reshape(B // 2, CP2)
    tid_pad = jnp.pad(t, ((0, 0), (0, TP - T))).reshape(B // 2, TP2)
    y = _cbow_sc(ctx_pad, tid_pad, in_emb, out_emb)
    return y[:, :T].reshape(B, 1, T)
